# Initial kernel scaffold; baseline (speedup 1.0000x reference)
import functools

import jax
import jax.numpy as jnp
from jax import lax
from jax.experimental import pallas as pl
from jax.experimental.pallas import tpu as pltpu
from jax.experimental.pallas import tpu_sc as plsc

N = 100000
E = 1600000
F_IN = 2
H = 64
D_E = 52
G = 64

NC, NS = 2, 16
NW = NC * NS
CH = 128
BE = 4096
EP = ((E + BE - 1) // BE) * BE
R = EP // CH
RG = R // NW
RS = R // NS
NDUMMY = 16
NACC = N + NDUMMY
ZR = NACC // NS
DR = N // NS
CG = 16
NBLK = 20
BN = N // NBLK

_P = jax.lax.Precision.HIGHEST

_mesh = plsc.VectorSubcoreMesh(core_axis_name="c", subcore_axis_name="s",
                               num_cores=NC, num_subcores=NS)


def _gather_body(src2d, x, xj, idxA, idxB, rowA, rowB, siA, siB, sgA, sgB, ssA, ssB):
    c = lax.axis_index("c")
    s = lax.axis_index("s")
    w = s * NC + c
    base = w * RG

    def loadI(r, idx_ref, si):
        pltpu.async_copy(src2d.at[base + r], idx_ref, si)

    def waitI(idx_ref, si):
        pltpu.make_async_copy(src2d.at[0], idx_ref, si).wait()

    def startG(idx_ref, row_ref, sg):
        pltpu.async_copy(x.at[idx_ref], row_ref, sg)

    def waitG(idx_ref, row_ref, sg):
        pltpu.make_async_copy(x.at[idx_ref], row_ref, sg).wait()

    def startS(r, row_ref, ss):
        pltpu.async_copy(row_ref, xj.at[pl.ds((base + r) * CH, CH)], ss)

    def waitS(row_ref, ss):
        pltpu.make_async_copy(row_ref, xj.at[pl.ds(0, CH)], ss).wait()

    loadI(0, idxA, siA)
    waitI(idxA, siA)
    startG(idxA, rowA, sgA)
    loadI(1, idxB, siB)

    def body(i, carry):
        r0 = 2 * i
        waitG(idxA, rowA, sgA)
        startS(r0, rowA, ssA)
        waitI(idxB, siB)
        startG(idxB, rowB, sgB)

        @pl.when(r0 + 2 < RG)
        def _():
            loadI(r0 + 2, idxA, siA)

        waitG(idxB, rowB, sgB)
        startS(r0 + 1, rowB, ssB)

        @pl.when(r0 + 3 < RG)
        def _():
            loadI(r0 + 3, idxB, siB)

        waitS(rowA, ssA)

        @pl.when(r0 + 2 < RG)
        def _():
            waitI(idxA, siA)
            startG(idxA, rowA, sgA)

        waitS(rowB, ssB)
        return carry

    lax.fori_loop(0, RG // 2, body, 0)
    waitG(idxA, rowA, sgA)
    startS(RG - 1, rowA, ssA)
    waitS(rowA, ssA)


_gather = functools.partial(
    pl.kernel,
    _gather_body,
    out_type=jax.ShapeDtypeStruct((EP, F_IN), jnp.float32),
    mesh=_mesh,
    scratch_types=[
        pltpu.VMEM((CH,), jnp.int32),
        pltpu.VMEM((CH,), jnp.int32),
        pltpu.VMEM((CH, F_IN), jnp.float32),
        pltpu.VMEM((CH, F_IN), jnp.float32),
        pltpu.SemaphoreType.DMA,
        pltpu.SemaphoreType.DMA,
        pltpu.SemaphoreType.DMA,
        pltpu.SemaphoreType.DMA,
        pltpu.SemaphoreType.DMA,
        pltpu.SemaphoreType.DMA,
    ],
)()


def _msg_body(ea_ref, xj_ref, w1_ref, b1m_ref, msg_ref):
    ea = ea_ref[...]
    th = lax.dot_general(ea, w1_ref[...], (((1,), (0,)), ((), ())),
                         preferred_element_type=jnp.float32, precision=_P)
    xj = xj_ref[...]
    x0 = xj[:, 0:1]
    x1 = xj[:, 1:2]
    b1m = b1m_ref[...]
    msg_ref[...] = (x0 * (th[:, :H] + b1m[0:1, :])
                    + x1 * (th[:, H:] + b1m[1:2, :]))


def _msg(ea, xj, w1, b1m):
    return pl.pallas_call(
        _msg_body,
        grid=(EP // BE,),
        in_specs=[
            pl.BlockSpec((BE, D_E), lambda i: (i, 0)),
            pl.BlockSpec((BE, F_IN), lambda i: (i, 0)),
            pl.BlockSpec((D_E, 2 * H), lambda i: (0, 0)),
            pl.BlockSpec((F_IN, H), lambda i: (0, 0)),
        ],
        out_specs=pl.BlockSpec((BE, H), lambda i: (i, 0)),
        out_shape=jax.ShapeDtypeStruct((EP, H), jnp.float32),
    )(ea, xj, w1, b1m)


def _scatter_body(dst2d, msg, zrows, aggr, idxA, idxB, bufA, bufB, acc,
                  siA, siB, smA, smB):
    c = lax.axis_index("c")
    s = lax.axis_index("s")

    for g_loc in range(2):
        col0 = (2 * c + g_loc) * CG

        pltpu.sync_copy(zrows, acc.at[pl.ds(s * ZR, ZR)])
        plsc.subcore_barrier()

        def load(r, idx_ref, buf_ref, si, sm):
            row = s * RS + r
            pltpu.async_copy(dst2d.at[row], idx_ref, si)
            pltpu.async_copy(msg.at[pl.ds(row * CH, CH), pl.ds(col0, CG)],
                             buf_ref, sm)

        def wait(idx_ref, buf_ref, si, sm):
            pltpu.make_async_copy(dst2d.at[0], idx_ref, si).wait()
            pltpu.make_async_copy(msg.at[pl.ds(0, CH), pl.ds(0, CG)],
                                  buf_ref, sm).wait()

        load(0, idxA, bufA, siA, smA)

        def body(i, carry):
            r0 = 2 * i
            load(r0 + 1, idxB, bufB, siB, smB)
            wait(idxA, bufA, siA, smA)
            pltpu.sync_copy(bufA, acc.at[idxA], add=True)

            @pl.when(r0 + 2 < RS)
            def _():
                load(r0 + 2, idxA, bufA, siA, smA)

            wait(idxB, bufB, siB, smB)
            pltpu.sync_copy(bufB, acc.at[idxB], add=True)
            return carry

        lax.fori_loop(0, RS // 2, body, 0)
        plsc.subcore_barrier()
        pltpu.sync_copy(acc.at[pl.ds(s * DR, DR)],
                        aggr.at[pl.ds(s * DR, DR), pl.ds(col0, CG)])
        plsc.subcore_barrier()


_scatter = functools.partial(
    pl.kernel,
    _scatter_body,
    out_type=jax.ShapeDtypeStruct((N, H), jnp.float32),
    mesh=_mesh,
    scratch_types=[
        pltpu.VMEM((CH,), jnp.int32),
        pltpu.VMEM((CH,), jnp.int32),
        pltpu.VMEM((CH, CG), jnp.float32),
        pltpu.VMEM((CH, CG), jnp.float32),
        pltpu.VMEM_SHARED((NACC, CG), jnp.float32),
        pltpu.SemaphoreType.DMA,
        pltpu.SemaphoreType.DMA,
        pltpu.SemaphoreType.DMA,
        pltpu.SemaphoreType.DMA,
    ],
)()


def _post_body(aggr_ref, x_ref, b3_ref, root_ref, cb_ref,
               fw1_ref, fb1_ref, fw2_ref, fb2_ref, fw3_ref, fb3_ref,
               fw4_ref, fb4_ref, y_ref, pool_acc, cnt_acc):
    i = pl.program_id(0)

    @pl.when(i == 0)
    def _():
        pool_acc[...] = jnp.zeros_like(pool_acc)
        cnt_acc[...] = jnp.zeros_like(cnt_acc)

    xr = lax.dot_general(x_ref[...], root_ref[...], (((1,), (0,)), ((), ())),
                         preferred_element_type=jnp.float32, precision=_P)
    out = jnp.maximum(aggr_ref[...] + xr + cb_ref[...], 0.0)
    b = b3_ref[0, 0, :]
    gids = lax.broadcasted_iota(jnp.int32, (G, BN), 0)
    oh = (b[None, :] == gids).astype(jnp.float32)
    pool_acc[...] += lax.dot_general(oh, out, (((1,), (0,)), ((), ())),
                                     preferred_element_type=jnp.float32,
                                     precision=_P)
    cnt_acc[...] += jnp.sum(oh, axis=1, keepdims=True)

    @pl.when(i == NBLK - 1)
    def _():
        pooled = pool_acc[...] / jnp.maximum(cnt_acc[...], 1.0)

        def lin(hh, w_ref, b_ref):
            return lax.dot_general(hh, w_ref[...], (((1,), (0,)), ((), ())),
                                   preferred_element_type=jnp.float32,
                                   precision=_P) + b_ref[...]

        h = jnp.maximum(lin(pooled, fw1_ref, fb1_ref), 0.0)
        h = jnp.maximum(lin(h, fw2_ref, fb2_ref), 0.0)
        h = jnp.maximum(lin(h, fw3_ref, fb3_ref), 0.0)
        y_ref[...] = lin(h, fw4_ref, fb4_ref)


def _post(aggr, x, b3, root, cb, fw1, fb1, fw2, fb2, fw3, fb3, fw4p, fb4p):
    def full(shape):
        nd = len(shape)
        return pl.BlockSpec(shape, lambda i, _nd=nd: (0,) * _nd)

    return pl.pallas_call(
        _post_body,
        grid=(NBLK,),
        in_specs=[
            pl.BlockSpec((BN, H), lambda i: (i, 0)),
            pl.BlockSpec((BN, F_IN), lambda i: (i, 0)),
            pl.BlockSpec((1, 1, BN), lambda i: (i, 0, 0)),
            full((F_IN, H)),
            full((1, H)),
            full((H, 128)),
            full((1, 128)),
            full((128, 256)),
            full((1, 256)),
            full((256, 128)),
            full((1, 128)),
            full((128, 128)),
            full((1, 128)),
        ],
        out_specs=pl.BlockSpec((G, 128), lambda i: (0, 0)),
        out_shape=jax.ShapeDtypeStruct((G, 128), jnp.float32),
        scratch_shapes=[pltpu.VMEM((G, H), jnp.float32),
                        pltpu.VMEM((G, 1), jnp.float32)],
    )(aggr, x, b3, root, cb, fw1, fb1, fw2, fb2, fw3, fb3, fw4p, fb4p)


def kernel(x, edge_index, edge_attr, batch, w1, b1, root, conv_bias,
           fw1, fb1, fw2, fb2, fw3, fb3, fw4, fb4):
    src = edge_index[0]
    dst = edge_index[1]
    pad = EP - E
    src_p = jnp.concatenate([src, jnp.zeros((pad,), jnp.int32)]).reshape(R, CH)
    dummy = N + (lax.iota(jnp.int32, pad) % NDUMMY)
    dst_p = jnp.concatenate([dst, dummy]).reshape(R, CH)
    zrows = jnp.zeros((ZR, CG), jnp.float32)
    b1m = b1.reshape(F_IN, H)

    xj = _gather(src_p, x)
    msg = _msg(edge_attr, xj, w1, b1m)
    aggr = _scatter(dst_p, msg, zrows)

    b3 = batch.reshape(NBLK, 1, BN)
    fw4p = jnp.pad(fw4, ((0, 0), (0, 127)))
    fb4p = jnp.pad(fb4, (0, 127)).reshape(1, 128)
    y = _post(aggr, x, b3, root, conv_bias.reshape(1, H),
              fw1, fb1.reshape(1, -1), fw2, fb2.reshape(1, -1),
              fw3, fb3.reshape(1, -1), fw4p, fb4p)
    return y[:, :1]


# speedup vs baseline: 3.0229x; 3.0229x over previous
import functools

import jax
import jax.numpy as jnp
from jax import lax
from jax.experimental import pallas as pl
from jax.experimental.pallas import tpu as pltpu
from jax.experimental.pallas import tpu_sc as plsc

N = 100000
E = 1600000
F_IN = 2
H = 64
D_E = 52
G = 64

NC, NS = 2, 16
NW = NC * NS
CH = 128
BE = 4096
EP = ((E + BE - 1) // BE) * BE
R = EP // CH
RG = R // NW
RS = R // NS
NDUMMY = 16
NACC = N + NDUMMY
ZR = NACC // NS
DR = N // NS
CG = 16
NBLK = 20
BN = N // NBLK

_P = jax.lax.Precision.HIGHEST

@functools.lru_cache(maxsize=1)
def _mesh():
    return plsc.VectorSubcoreMesh(core_axis_name="c", subcore_axis_name="s",
                                  num_cores=NC, num_subcores=NS)


def _gather_body(src2d, x, xj, idxA, idxB, rowA, rowB, siA, siB, sgA, sgB, ssA, ssB):
    c = lax.axis_index("c")
    s = lax.axis_index("s")
    w = s * NC + c
    base = w * RG

    def loadI(r, idx_ref, si):
        pltpu.async_copy(src2d.at[base + r], idx_ref, si)

    def waitI(idx_ref, si):
        pltpu.make_async_copy(src2d.at[0], idx_ref, si).wait()

    def startG(idx_ref, row_ref, sg):
        pltpu.async_copy(x.at[idx_ref], row_ref, sg)

    def waitG(idx_ref, row_ref, sg):
        pltpu.make_async_copy(x.at[idx_ref], row_ref, sg).wait()

    def startS(r, row_ref, ss):
        pltpu.async_copy(row_ref, xj.at[pl.ds((base + r) * CH, CH)], ss)

    def waitS(row_ref, ss):
        pltpu.make_async_copy(row_ref, xj.at[pl.ds(0, CH)], ss).wait()

    loadI(0, idxA, siA)
    waitI(idxA, siA)
    startG(idxA, rowA, sgA)
    loadI(1, idxB, siB)

    def body(i, carry):
        r0 = 2 * i
        waitG(idxA, rowA, sgA)
        startS(r0, rowA, ssA)
        waitI(idxB, siB)
        startG(idxB, rowB, sgB)

        @pl.when(r0 + 2 < RG)
        def _():
            loadI(r0 + 2, idxA, siA)

        waitG(idxB, rowB, sgB)
        startS(r0 + 1, rowB, ssB)

        @pl.when(r0 + 3 < RG)
        def _():
            loadI(r0 + 3, idxB, siB)

        waitS(rowA, ssA)

        @pl.when(r0 + 2 < RG)
        def _():
            waitI(idxA, siA)
            startG(idxA, rowA, sgA)

        waitS(rowB, ssB)
        return carry

    lax.fori_loop(0, RG // 2, body, 0)
    waitG(idxA, rowA, sgA)
    startS(RG - 1, rowA, ssA)
    waitS(rowA, ssA)


def _gather(src2d, x16):
    k = pl.kernel(
        _gather_body,
        out_type=jax.ShapeDtypeStruct((EP, 16), jnp.float32),
        mesh=_mesh(),
        compiler_params=pltpu.CompilerParams(use_tc_tiling_on_sc=False),
        scratch_types=[
            pltpu.VMEM((CH,), jnp.int32),
            pltpu.VMEM((CH,), jnp.int32),
            pltpu.VMEM((CH, 16), jnp.float32),
            pltpu.VMEM((CH, 16), jnp.float32),
            pltpu.SemaphoreType.DMA,
            pltpu.SemaphoreType.DMA,
            pltpu.SemaphoreType.DMA,
            pltpu.SemaphoreType.DMA,
            pltpu.SemaphoreType.DMA,
            pltpu.SemaphoreType.DMA,
        ],
    )
    return k(src2d, x16)


def _msg_body(ea_ref, xj_ref, w1_ref, b1m_ref, msg_ref):
    ea = ea_ref[...]
    th = lax.dot_general(ea, w1_ref[...], (((1,), (0,)), ((), ())),
                         preferred_element_type=jnp.float32, precision=_P)
    xj = xj_ref[...]
    x0 = xj[:, 0:1]
    x1 = xj[:, 1:2]
    b1m = b1m_ref[...]
    msg_ref[...] = (x0 * (th[:, :H] + b1m[0:1, :])
                    + x1 * (th[:, H:] + b1m[1:2, :]))


def _msg(ea, xj, w1, b1m):
    return pl.pallas_call(
        _msg_body,
        grid=(EP // BE,),
        in_specs=[
            pl.BlockSpec((BE, D_E), lambda i: (i, 0)),
            pl.BlockSpec((BE, 16), lambda i: (i, 0)),
            pl.BlockSpec((D_E, 2 * H), lambda i: (0, 0)),
            pl.BlockSpec((F_IN, H), lambda i: (0, 0)),
        ],
        out_specs=pl.BlockSpec((BE, H), lambda i: (i, 0)),
        out_shape=jax.ShapeDtypeStruct((EP, H), jnp.float32),
    )(ea, xj, w1, b1m)


def _scatter_body(dst2d, msg, zrows, aggr, idxA, idxB, bufA, bufB, acc,
                  siA, siB, smA, smB):
    c = lax.axis_index("c")
    s = lax.axis_index("s")

    for g_loc in range(2):
        col0 = (2 * c + g_loc) * CG

        pltpu.sync_copy(zrows, acc.at[pl.ds(s * ZR, ZR)])
        plsc.subcore_barrier()

        def load(r, idx_ref, buf_ref, si, sm):
            row = s * RS + r
            pltpu.async_copy(dst2d.at[row], idx_ref, si)
            pltpu.async_copy(msg.at[pl.ds(row * CH, CH), pl.ds(col0, CG)],
                             buf_ref, sm)

        def wait(idx_ref, buf_ref, si, sm):
            pltpu.make_async_copy(dst2d.at[0], idx_ref, si).wait()
            pltpu.make_async_copy(msg.at[pl.ds(0, CH), pl.ds(0, CG)],
                                  buf_ref, sm).wait()

        load(0, idxA, bufA, siA, smA)

        def body(i, carry):
            r0 = 2 * i
            load(r0 + 1, idxB, bufB, siB, smB)
            wait(idxA, bufA, siA, smA)
            pltpu.sync_copy(bufA, acc.at[idxA], add=True)

            @pl.when(r0 + 2 < RS)
            def _():
                load(r0 + 2, idxA, bufA, siA, smA)

            wait(idxB, bufB, siB, smB)
            pltpu.sync_copy(bufB, acc.at[idxB], add=True)
            return carry

        lax.fori_loop(0, RS // 2, body, 0)
        plsc.subcore_barrier()
        pltpu.sync_copy(acc.at[pl.ds(s * DR, DR)],
                        aggr.at[pl.ds(s * DR, DR), pl.ds(col0, CG)])
        plsc.subcore_barrier()


def _scatter(dst2d, msg, zrows):
    k = pl.kernel(
        _scatter_body,
        out_type=jax.ShapeDtypeStruct((N, H), jnp.float32),
        mesh=_mesh(),
        compiler_params=pltpu.CompilerParams(use_tc_tiling_on_sc=False),
        scratch_types=[
            pltpu.VMEM((CH,), jnp.int32),
            pltpu.VMEM((CH,), jnp.int32),
            pltpu.VMEM((CH, CG), jnp.float32),
            pltpu.VMEM((CH, CG), jnp.float32),
            pltpu.VMEM_SHARED((NACC, CG), jnp.float32),
            pltpu.SemaphoreType.DMA,
            pltpu.SemaphoreType.DMA,
            pltpu.SemaphoreType.DMA,
            pltpu.SemaphoreType.DMA,
        ],
    )
    return k(dst2d, msg, zrows)


def _post_body(aggr_ref, x_ref, b3_ref, root_ref, cb_ref,
               fw1_ref, fb1_ref, fw2_ref, fb2_ref, fw3_ref, fb3_ref,
               fw4_ref, fb4_ref, y_ref, pool_acc, cnt_acc):
    i = pl.program_id(0)

    @pl.when(i == 0)
    def _():
        pool_acc[...] = jnp.zeros_like(pool_acc)
        cnt_acc[...] = jnp.zeros_like(cnt_acc)

    xr = lax.dot_general(x_ref[...], root_ref[...], (((1,), (0,)), ((), ())),
                         preferred_element_type=jnp.float32, precision=_P)
    out = jnp.maximum(aggr_ref[...] + xr + cb_ref[...], 0.0)
    b = b3_ref[0, 0, :]
    gids = lax.broadcasted_iota(jnp.int32, (G, BN), 0)
    oh = (b[None, :] == gids).astype(jnp.float32)
    pool_acc[...] += lax.dot_general(oh, out, (((1,), (0,)), ((), ())),
                                     preferred_element_type=jnp.float32,
                                     precision=_P)
    cnt_acc[...] += jnp.sum(oh, axis=1, keepdims=True)

    @pl.when(i == NBLK - 1)
    def _():
        pooled = pool_acc[...] / jnp.maximum(cnt_acc[...], 1.0)

        def lin(hh, w_ref, b_ref):
            return lax.dot_general(hh, w_ref[...], (((1,), (0,)), ((), ())),
                                   preferred_element_type=jnp.float32,
                                   precision=_P) + b_ref[...]

        h = jnp.maximum(lin(pooled, fw1_ref, fb1_ref), 0.0)
        h = jnp.maximum(lin(h, fw2_ref, fb2_ref), 0.0)
        h = jnp.maximum(lin(h, fw3_ref, fb3_ref), 0.0)
        y_ref[...] = lin(h, fw4_ref, fb4_ref)


def _post(aggr, x, b3, root, cb, fw1, fb1, fw2, fb2, fw3, fb3, fw4p, fb4p):
    def full(shape):
        nd = len(shape)
        return pl.BlockSpec(shape, lambda i, _nd=nd: (0,) * _nd)

    return pl.pallas_call(
        _post_body,
        grid=(NBLK,),
        in_specs=[
            pl.BlockSpec((BN, H), lambda i: (i, 0)),
            pl.BlockSpec((BN, F_IN), lambda i: (i, 0)),
            pl.BlockSpec((1, 1, BN), lambda i: (i, 0, 0)),
            full((F_IN, H)),
            full((1, H)),
            full((H, 128)),
            full((1, 128)),
            full((128, 256)),
            full((1, 256)),
            full((256, 128)),
            full((1, 128)),
            full((128, 128)),
            full((1, 128)),
        ],
        out_specs=pl.BlockSpec((G, 128), lambda i: (0, 0)),
        out_shape=jax.ShapeDtypeStruct((G, 128), jnp.float32),
        scratch_shapes=[pltpu.VMEM((G, H), jnp.float32),
                        pltpu.VMEM((G, 1), jnp.float32)],
    )(aggr, x, b3, root, cb, fw1, fb1, fw2, fb2, fw3, fb3, fw4p, fb4p)


def kernel(x, edge_index, edge_attr, batch, w1, b1, root, conv_bias,
           fw1, fb1, fw2, fb2, fw3, fb3, fw4, fb4):
    src = edge_index[0]
    dst = edge_index[1]
    pad = EP - E
    src_p = jnp.concatenate([src, jnp.zeros((pad,), jnp.int32)]).reshape(R, CH)
    dummy = N + (lax.iota(jnp.int32, pad) % NDUMMY)
    dst_p = jnp.concatenate([dst, dummy]).reshape(R, CH)
    zrows = jnp.zeros((ZR, CG), jnp.float32)
    b1m = b1.reshape(F_IN, H)

    x16 = jnp.pad(x, ((0, 0), (0, 16 - F_IN)))
    xj = _gather(src_p, x16)
    msg = _msg(edge_attr, xj, w1, b1m)
    aggr = _scatter(dst_p, msg, zrows)

    b3 = batch.reshape(NBLK, 1, BN)
    fw4p = jnp.pad(fw4, ((0, 0), (0, 127)))
    fb4p = jnp.pad(fb4, (0, 127)).reshape(1, 128)
    y = _post(aggr, x, b3, root, conv_bias.reshape(1, H),
              fw1, fb1.reshape(1, -1), fw2, fb2.reshape(1, -1),
              fw3, fb3.reshape(1, -1), fw4p, fb4p)
    return y[:, :1]
